# baseline (speedup 1.0000x reference)
import functools

import jax
import jax.numpy as jnp
from jax.experimental import pallas as pl


def _matmul_bias_kernel(x_ref, w_ref, b_ref, o_ref):
    acc = jax.lax.dot_general(
        x_ref[...],
        w_ref[...],
        dimension_numbers=(((1,), (1,)), ((), ())),
        preferred_element_type=jnp.float32,
    )

    @pl.when(pl.program_id(0) == 0)
    def _():
        o_ref[...] = acc + b_ref[...]

    @pl.when(pl.program_id(0) != 0)
    def _():
        o_ref[...] += acc


def kernel(input, mems_x, mems_y, W, b):
    del mems_x, mems_y
    n_b, d = input.shape
    n_c = W.shape[0]
    k_steps = 2
    dk = d // k_steps
    return pl.pallas_call(
        _matmul_bias_kernel,
        grid=(k_steps,),
        in_specs=[
            pl.BlockSpec((n_b, dk), lambda k: (0, k)),
            pl.BlockSpec((n_c, dk), lambda k: (0, k)),
            pl.BlockSpec((1, n_c), lambda k: (0, 0)),
        ],
        out_specs=pl.BlockSpec((n_b, n_c), lambda k: (0, 0)),
        out_shape=jax.ShapeDtypeStruct((n_b, n_c), jnp.float32),
    )(input, W, b.reshape(1, n_c))


# speedup vs baseline: 1.0189x; 1.0189x over previous
import jax
import jax.numpy as jnp
from jax.experimental import pallas as pl


def _matmul_bias_kernel(x_ref, w_ref, b_ref, o_ref):
    o_ref[...] = jax.lax.dot_general(
        x_ref[...],
        w_ref[...],
        dimension_numbers=(((1,), (1,)), ((), ())),
        preferred_element_type=jnp.float32,
    ) + b_ref[...]


def kernel(input, mems_x, mems_y, W, b):
    del mems_x, mems_y
    n_b, _ = input.shape
    n_c = W.shape[0]
    return pl.pallas_call(
        _matmul_bias_kernel,
        out_shape=jax.ShapeDtypeStruct((n_b, n_c), jnp.float32),
    )(input, W, b.reshape(1, n_c))


# speedup vs baseline: 1.0252x; 1.0062x over previous
import jax
import jax.numpy as jnp
from jax.experimental import pallas as pl
from jax.experimental.pallas import tpu as pltpu


def _matmul_bias_kernel(x_hbm, w_hbm, b_hbm, o_ref, x_v, w_v, b_v, sems):
    cpx = pltpu.make_async_copy(x_hbm, x_v, sems.at[0])
    cpw = pltpu.make_async_copy(w_hbm, w_v, sems.at[1])
    cpb = pltpu.make_async_copy(b_hbm, b_v, sems.at[2])
    cpx.start()
    cpw.start()
    cpb.start()
    cpx.wait()
    cpw.wait()
    cpb.wait()
    o_ref[...] = jax.lax.dot_general(
        x_v[...],
        w_v[...],
        dimension_numbers=(((1,), (1,)), ((), ())),
        preferred_element_type=jnp.float32,
    ) + b_v[...]


def kernel(input, mems_x, mems_y, W, b):
    del mems_x, mems_y
    n_b, d = input.shape
    n_c = W.shape[0]
    return pl.pallas_call(
        _matmul_bias_kernel,
        in_specs=[
            pl.BlockSpec(memory_space=pl.ANY),
            pl.BlockSpec(memory_space=pl.ANY),
            pl.BlockSpec(memory_space=pl.ANY),
        ],
        out_shape=jax.ShapeDtypeStruct((n_b, n_c), jnp.float32),
        scratch_shapes=[
            pltpu.VMEM((n_b, d), jnp.float32),
            pltpu.VMEM((n_c, d), jnp.float32),
            pltpu.VMEM((1, n_c), jnp.float32),
            pltpu.SemaphoreType.DMA((3,)),
        ],
    )(input, W, b.reshape(1, n_c))
